# Initial kernel scaffold; baseline (speedup 1.0000x reference)
import functools

import jax
import jax.numpy as jnp
from jax import lax
from jax.experimental import pallas as pl
from jax.experimental.pallas import tpu as pltpu
from jax.experimental.pallas import tpu_sc as plsc

N = 10000
E = 320000
D = 128
NC = 2
NS = 16
NW = NC * NS
EPW = E // NW
K = 80
NCHUNK = EPW // K
RPT = N // NS


def _sc_partial(data, edge_index, zeros):
    mesh = plsc.VectorSubcoreMesh(
        core_axis_name="c", subcore_axis_name="s", num_cores=NC
    )

    @functools.partial(
        pl.kernel,
        out_type=jax.ShapeDtypeStruct((NC, N, D), jnp.float32),
        mesh=mesh,
        scratch_types=[
            pltpu.VMEM_SHARED((N, D), jnp.float32),
            pltpu.VMEM((K,), jnp.int32),
            pltpu.VMEM((K,), jnp.int32),
            pltpu.VMEM((K, D), jnp.float32),
            pltpu.SemaphoreType.DMA,
        ],
    )
    def k(data_hbm, edge_hbm, zero_hbm, out_hbm, acc, src_v, dst_v, rows_v, sem):
        c = lax.axis_index("c")
        s = lax.axis_index("s")
        wid = s * NC + c

        pltpu.sync_copy(
            zero_hbm.at[pl.ds(s * RPT, RPT)], acc.at[pl.ds(s * RPT, RPT)]
        )
        plsc.subcore_barrier()

        base0 = wid * EPW

        @pl.loop(0, NCHUNK)
        def _chunk(i):
            base = base0 + i * K
            pltpu.sync_copy(edge_hbm.at[0, pl.ds(base, K)], src_v)
            pltpu.sync_copy(edge_hbm.at[1, pl.ds(base, K)], dst_v)
            pltpu.async_copy(data_hbm.at[src_v], rows_v, sem).wait()
            pltpu.sync_copy(rows_v, acc.at[dst_v], add=True)

        plsc.subcore_barrier()
        pltpu.sync_copy(
            acc.at[pl.ds(s * RPT, RPT)], out_hbm.at[c, pl.ds(s * RPT, RPT)]
        )

    return k(data, edge_index, zeros)


def _combine(partial):
    def body(p_ref, o_ref):
        o_ref[...] = p_ref[0] + p_ref[1]

    return pl.pallas_call(
        body,
        out_shape=jax.ShapeDtypeStruct((N, D), jnp.float32),
        grid=(10,),
        in_specs=[pl.BlockSpec((2, 1000, D), lambda i: (0, i, 0))],
        out_specs=pl.BlockSpec((1000, D), lambda i: (i, 0)),
    )(partial)


@jax.jit
def kernel(data, edge_index):
    zeros = jnp.zeros((N, D), jnp.float32)
    partial = _sc_partial(data, edge_index, zeros)
    return _combine(partial)


# speedup vs baseline: 5.4783x; 5.4783x over previous
import functools

import jax
import jax.numpy as jnp
from jax import lax
from jax.experimental import pallas as pl
from jax.experimental.pallas import tpu as pltpu
from jax.experimental.pallas import tpu_sc as plsc

N = 10000
E = 320000
D = 128
NC = 2
NS = 16
NW = NC * NS
EPW = E // NW
K = 80
NCHUNK = EPW // K
RPT = 624
REM = N - RPT * NS


def _sc_partial(data, edge_index, zeros):
    mesh = plsc.VectorSubcoreMesh(
        core_axis_name="c", subcore_axis_name="s", num_cores=NC
    )

    @functools.partial(
        pl.kernel,
        out_type=jax.ShapeDtypeStruct((NC, N, D), jnp.float32),
        mesh=mesh,
        scratch_types=[
            pltpu.VMEM_SHARED((N, D), jnp.float32),
            pltpu.VMEM((K,), jnp.int32),
            pltpu.VMEM((K,), jnp.int32),
            pltpu.VMEM((K, D), jnp.float32),
            pltpu.SemaphoreType.DMA,
        ],
    )
    def k(data_hbm, se_hbm, de_hbm, zero_hbm, out_hbm, acc, src_v, dst_v, rows_v, sem):
        c = lax.axis_index("c")
        s = lax.axis_index("s")
        wid = s * NC + c

        pltpu.sync_copy(
            zero_hbm.at[pl.ds(s * RPT, RPT)], acc.at[pl.ds(s * RPT, RPT)]
        )

        @pl.when(s == 0)
        def _zero_rem():
            pltpu.sync_copy(
                zero_hbm.at[pl.ds(RPT * NS, REM)], acc.at[pl.ds(RPT * NS, REM)]
            )

        plsc.subcore_barrier()

        base0 = wid * EPW

        @pl.loop(0, NCHUNK)
        def _chunk(i):
            base = base0 + i * K
            pltpu.sync_copy(se_hbm.at[pl.ds(base, K)], src_v)
            pltpu.sync_copy(de_hbm.at[pl.ds(base, K)], dst_v)
            pltpu.async_copy(data_hbm.at[src_v], rows_v, sem).wait()
            pltpu.sync_copy(rows_v, acc.at[dst_v], add=True)

        plsc.subcore_barrier()
        pltpu.sync_copy(
            acc.at[pl.ds(s * RPT, RPT)], out_hbm.at[c, pl.ds(s * RPT, RPT)]
        )

        @pl.when(s == 0)
        def _flush_rem():
            pltpu.sync_copy(
                acc.at[pl.ds(RPT * NS, REM)], out_hbm.at[c, pl.ds(RPT * NS, REM)]
            )

    return k(data, edge_index[0], edge_index[1], zeros)


def _combine(partial):
    def body(p_ref, o_ref):
        o_ref[...] = p_ref[0] + p_ref[1]

    return pl.pallas_call(
        body,
        out_shape=jax.ShapeDtypeStruct((N, D), jnp.float32),
        grid=(10,),
        in_specs=[pl.BlockSpec((2, 1000, D), lambda i: (0, i, 0))],
        out_specs=pl.BlockSpec((1000, D), lambda i: (i, 0)),
    )(partial)


@jax.jit
def kernel(data, edge_index):
    zeros = jnp.zeros((N, D), jnp.float32)
    partial = _sc_partial(data, edge_index, zeros)
    return _combine(partial)
